# baseline (speedup 1.0000x reference)
import functools

import jax
import jax.numpy as jnp
from jax import lax
from jax.experimental import pallas as pl
from jax.experimental.pallas import tpu as pltpu
from jax.experimental.pallas import tpu_sc as plsc

NSEG = 256
NEG = -1e30
WIN = 64


def _main_body(x_ref, u_ref, w1_ref, b1_ref, w2_ref, b2_ref, batch_ref,
               bounds_ref,
               s_ref, m_ref, d_ref, pooled_ref,
               mx_s, d_s, acc_s, uw_s):
    i = pl.program_id(0)
    nsteps = pl.num_programs(0)
    nf = x_ref.shape[1]

    @pl.when(i == 0)
    def _init():
        mx_s[0, 0] = NEG
        d_s[...] = jnp.zeros_like(d_s)
        acc_s[...] = jnp.zeros_like(acc_s)
        uw_s[...] = jnp.dot(u_ref[...], w1_ref[nf:, :],
                            preferred_element_type=jnp.float32) + b1_ref[...]

    B = x_ref.shape[0]
    H = B // 2
    xa = x_ref[0:H, :]
    xb = x_ref[H:, :]
    brow_a = batch_ref[0, 0:1, :]
    brow_b = batch_ref[0, 1:2, :]

    bf_a = bounds_ref[0, 0, 0]
    bl_a = bounds_ref[0, 0, 1]
    bf_b = bounds_ref[0, 1, 0]
    bl_b = bounds_ref[0, 1, 1]
    bfa_a = jnp.minimum((bf_a // 8) * 8, NSEG - WIN)
    bfa_b = jnp.minimum((bf_b // 8) * 8, NSEG - WIN)
    narrow = jnp.logical_and(bl_a - bfa_a < WIN, bl_b - bfa_b < WIN)

    def score_half(wseg, base, xh, browh):
        at_cmp = (lax.broadcasted_iota(jnp.int32, (wseg, H), 0)
                  == browh - base)
        at_f = at_cmp.astype(jnp.float32)
        ht = lax.dot_general(w1_ref[:nf, :], xh, (((0,), (1,)), ((), ())),
                             preferred_element_type=jnp.float32)
        uww = uw_s[pl.ds(base, wseg), :]
        ht = ht + lax.dot_general(uww, at_f, (((0,), (0,)), ((), ())),
                                  preferred_element_type=jnp.float32)
        ht = jnp.maximum(ht, 0.0)
        st = lax.dot_general(w2_ref[...], ht, (((0,), (0,)), ((), ())),
                             preferred_element_type=jnp.float32)
        return at_cmp, st + b2_ref[0, 0]

    ones_col = jnp.ones((H, 1), dtype=jnp.float32)

    def accum_half(wseg, base, xh, at_cmp, st, m_new):
        e_row = jnp.exp(st - m_new)
        at_e = jnp.where(at_cmp, e_row, 0.0)
        p = jnp.dot(at_e, ones_col, preferred_element_type=jnp.float32)
        part = jnp.dot(at_e, xh, preferred_element_type=jnp.float32)
        d_s[pl.ds(base, wseg), :] = d_s[pl.ds(base, wseg), :] + p
        acc_s[pl.ds(base, wseg), :] = acc_s[pl.ds(base, wseg), :] + part

    def run(wseg, base_a, base_b):
        at_a, st_a = score_half(wseg, base_a, xa, brow_a)
        at_b, st_b = score_half(wseg, base_b, xb, brow_b)
        s_ref[0, 0, :] = st_a[0, :]
        s_ref[0, 1, :] = st_b[0, :]
        m_old = mx_s[0, 0]
        m_new = jnp.maximum(jnp.maximum(m_old, jnp.max(st_a)),
                            jnp.max(st_b))
        r = jnp.exp(m_old - m_new)
        mx_s[0, 0] = m_new
        d_s[...] = d_s[...] * r
        acc_s[...] = acc_s[...] * r
        accum_half(wseg, base_a, xa, at_a, st_a, m_new)
        accum_half(wseg, base_b, xb, at_b, st_b, m_new)

    @pl.when(narrow)
    def _narrow():
        run(WIN, bfa_a, bfa_b)

    @pl.when(jnp.logical_not(narrow))
    def _wide():
        run(NSEG, 0, 0)

    @pl.when(i == nsteps - 1)
    def _fin():
        d_fin = d_s[...]
        m_ref[...] = jnp.full_like(m_ref, mx_s[0, 0])
        d_ref[...] = d_fin
        pooled_ref[...] = jnp.where(d_fin > 0.0, acc_s[...] / d_fin, 0.0)


def _make_sc_attn(n, nw, rows):
    tail = n - (nw - 1) * rows
    assert 0 < tail <= rows and tail % 16 == 0
    nfull_full, rem_full = rows // 64, (rows % 64) // 16
    nfull_last, rem_last = tail // 64, (tail % 64) // 16
    mesh = plsc.VectorSubcoreMesh(core_axis_name="c", subcore_axis_name="s")

    @functools.partial(
        pl.kernel,
        mesh=mesh,
        compiler_params=pltpu.CompilerParams(needs_layout_passes=False),
        out_type=jax.ShapeDtypeStruct((n,), jnp.float32),
        scratch_types=[
            pltpu.VMEM((rows,), jnp.float32),
            pltpu.VMEM((rows,), jnp.int32),
            pltpu.VMEM((NSEG,), jnp.float32),
            pltpu.VMEM((NSEG,), jnp.float32),
            pltpu.VMEM((rows,), jnp.float32),
            pltpu.SemaphoreType.DMA,
        ],
    )
    def sc_attn(s_hbm, b_hbm, m_hbm, d_hbm, out_hbm, s_v, b_v, m_v, d_v, a_v,
                sem):
        wid = lax.axis_index("s") * 2 + lax.axis_index("c")
        base = wid * rows
        last = wid == nw - 1

        c3 = pltpu.async_copy(m_hbm, m_v, sem)
        c4 = pltpu.async_copy(d_hbm, d_v, sem)

        @pl.when(last)
        def _ld_last():
            c1 = pltpu.async_copy(s_hbm.at[pl.ds(base, tail)],
                                  s_v.at[pl.ds(0, tail)], sem)
            c2 = pltpu.async_copy(b_hbm.at[pl.ds(base, tail)],
                                  b_v.at[pl.ds(0, tail)], sem)
            c1.wait()
            c2.wait()

        @pl.when(jnp.logical_not(last))
        def _ld_full():
            c1 = pltpu.async_copy(s_hbm.at[pl.ds(base, rows)], s_v, sem)
            c2 = pltpu.async_copy(b_hbm.at[pl.ds(base, rows)], b_v, sem)
            c1.wait()
            c2.wait()

        c3.wait()
        c4.wait()

        msplat = m_v[pl.ds(0, 16)]
        for k in range(NSEG // 16):
            d_v[pl.ds(k * 16, 16)] = 1.0 / d_v[pl.ds(k * 16, 16)]

        nfull = jnp.where(last, nfull_last, nfull_full)
        rem = jnp.where(last, rem_last, rem_full)

        def norm16(off):
            idx = b_v[pl.ds(off, 16)]
            sv = s_v[pl.ds(off, 16)]
            iv = plsc.load_gather(d_v, [idx])
            a_v[pl.ds(off, 16)] = jnp.exp(sv - msplat) * iv

        def body(j, carry):
            for k in range(4):
                norm16(j * 64 + k * 16)
            return carry

        lax.fori_loop(0, nfull, body, 0)
        roff = nfull * 64

        def body_tail(t, carry):
            norm16(roff + t * 16)
            return carry

        lax.fori_loop(0, rem, body_tail, 0)

        @pl.when(last)
        def _st_last():
            pltpu.sync_copy(a_v.at[pl.ds(0, tail)],
                            out_hbm.at[pl.ds(base, tail)])

        @pl.when(jnp.logical_not(last))
        def _st_full():
            pltpu.sync_copy(a_v, out_hbm.at[pl.ds(base, rows)])

    return sc_attn


def kernel(x, u, W1, b1, W2, b2, batch):
    n, nf = x.shape
    hid = W1.shape[1]
    batch = batch.astype(jnp.int32)

    B = 20000
    H = B // 2
    nsteps = n // B
    assert nsteps * B == n

    batch3 = batch.reshape(nsteps, 2, H)
    bh = batch3[:, :, 0]
    bl = batch3[:, :, H - 1]
    bounds = jnp.stack([bh, bl], axis=2)
    s3, m, d, pooled = pl.pallas_call(
        _main_body,
        grid=(nsteps,),
        in_specs=[
            pl.BlockSpec((B, nf), lambda i: (i, 0)),
            pl.BlockSpec((NSEG, nf), lambda i: (0, 0)),
            pl.BlockSpec((nf + nf, hid), lambda i: (0, 0)),
            pl.BlockSpec((1, hid), lambda i: (0, 0)),
            pl.BlockSpec((hid, 1), lambda i: (0, 0)),
            pl.BlockSpec((1, 1), lambda i: (0, 0)),
            pl.BlockSpec((1, 2, H), lambda i: (i, 0, 0)),
            pl.BlockSpec((1, 2, 2), lambda i: (i, 0, 0),
                         memory_space=pltpu.SMEM),
        ],
        out_specs=[
            pl.BlockSpec((1, 2, H), lambda i: (i, 0, 0)),
            pl.BlockSpec((1, NSEG), lambda i: (0, 0)),
            pl.BlockSpec((NSEG, 1), lambda i: (0, 0)),
            pl.BlockSpec((NSEG, nf), lambda i: (0, 0)),
        ],
        out_shape=[
            jax.ShapeDtypeStruct((nsteps, 2, H), jnp.float32),
            jax.ShapeDtypeStruct((1, NSEG), jnp.float32),
            jax.ShapeDtypeStruct((NSEG, 1), jnp.float32),
            jax.ShapeDtypeStruct((NSEG, nf), jnp.float32),
        ],
        scratch_shapes=[
            pltpu.SMEM((1, 1), jnp.float32),
            pltpu.VMEM((NSEG, 1), jnp.float32),
            pltpu.VMEM((NSEG, nf), jnp.float32),
            pltpu.VMEM((NSEG, hid), jnp.float32),
        ],
        compiler_params=pltpu.CompilerParams(
            dimension_semantics=("arbitrary",)),
    )(x, u, W1, b1.reshape(1, hid), W2, b2.reshape(1, 1), batch3, bounds)

    nw = 32
    rows = -(-n // (nw * 64)) * 64
    sc_attn = _make_sc_attn(n, nw, rows)
    attn = sc_attn(s3.reshape(n), batch, m.reshape(NSEG), d.reshape(NSEG))
    return pooled, attn


# speedup vs baseline: 1.0117x; 1.0117x over previous
import functools

import jax
import jax.numpy as jnp
from jax import lax
from jax.experimental import pallas as pl
from jax.experimental.pallas import tpu as pltpu
from jax.experimental.pallas import tpu_sc as plsc

NSEG = 256
NEG = -1e30
WIN = 64


def _main_body(x_ref, u_ref, w1_ref, b1_ref, w2_ref, b2_ref, batch_ref,
               bounds_ref,
               s_ref, m_ref, d_ref, pooled_ref,
               mx_s, d_s, acc_s, uw_s):
    i = pl.program_id(0)
    nsteps = pl.num_programs(0)
    nf = x_ref.shape[1]

    @pl.when(i == 0)
    def _init():
        mx_s[0, 0] = NEG
        d_s[...] = jnp.zeros_like(d_s)
        acc_s[...] = jnp.zeros_like(acc_s)
        uw_s[...] = jnp.dot(u_ref[...], w1_ref[nf:, :],
                            preferred_element_type=jnp.float32) + b1_ref[...]

    B = x_ref.shape[0]
    H = B // 2
    xa = x_ref[0:H, :]
    xb = x_ref[H:, :]
    brow_a = batch_ref[0, 0:1, :]
    brow_b = batch_ref[0, 1:2, :]

    bf_a = bounds_ref[0, 0, 0]
    bl_a = bounds_ref[0, 0, 1]
    bf_b = bounds_ref[0, 1, 0]
    bl_b = bounds_ref[0, 1, 1]
    bfa_a = jnp.minimum((bf_a // 8) * 8, NSEG - WIN)
    bfa_b = jnp.minimum((bf_b // 8) * 8, NSEG - WIN)
    narrow = jnp.logical_and(bl_a - bfa_a < WIN, bl_b - bfa_b < WIN)

    def score_half(wseg, base, xh, browh):
        at_cmp = (lax.broadcasted_iota(jnp.int32, (wseg, H), 0)
                  == browh - base)
        at_f = at_cmp.astype(jnp.float32)
        ht = lax.dot_general(w1_ref[:nf, :], xh, (((0,), (1,)), ((), ())),
                             preferred_element_type=jnp.float32)
        uww = uw_s[pl.ds(base, wseg), :]
        ht = ht + lax.dot_general(uww, at_f, (((0,), (0,)), ((), ())),
                                  preferred_element_type=jnp.float32)
        ht = jnp.maximum(ht, 0.0)
        st = lax.dot_general(w2_ref[...], ht, (((0,), (0,)), ((), ())),
                             preferred_element_type=jnp.float32)
        return at_cmp, st + b2_ref[0, 0]

    ones_col = jnp.ones((H, 1), dtype=jnp.float32)

    def accum_half(wseg, base, xh, at_cmp, st, m_new):
        e_row = jnp.exp(st - m_new)
        at_e = jnp.where(at_cmp, e_row, 0.0)
        p = jnp.dot(at_e, ones_col, preferred_element_type=jnp.float32)
        part = jnp.dot(at_e, xh, preferred_element_type=jnp.float32)
        d_s[pl.ds(base, wseg), :] = d_s[pl.ds(base, wseg), :] + p
        acc_s[pl.ds(base, wseg), :] = acc_s[pl.ds(base, wseg), :] + part

    def run(wseg, base_a, base_b):
        at_a, st_a = score_half(wseg, base_a, xa, brow_a)
        at_b, st_b = score_half(wseg, base_b, xb, brow_b)
        s_ref[0, 0, :] = st_a[0, :]
        s_ref[0, 1, :] = st_b[0, :]
        m_old = mx_s[0, 0]
        m_new = jnp.maximum(jnp.maximum(m_old, jnp.max(st_a)),
                            jnp.max(st_b))
        r = jnp.exp(m_old - m_new)
        mx_s[0, 0] = m_new
        d_s[...] = d_s[...] * r
        acc_s[...] = acc_s[...] * r
        accum_half(wseg, base_a, xa, at_a, st_a, m_new)
        accum_half(wseg, base_b, xb, at_b, st_b, m_new)

    @pl.when(narrow)
    def _narrow():
        run(WIN, bfa_a, bfa_b)

    @pl.when(jnp.logical_not(narrow))
    def _wide():
        run(NSEG, 0, 0)

    @pl.when(i == nsteps - 1)
    def _fin():
        d_fin = d_s[...]
        m_ref[...] = jnp.full_like(m_ref, mx_s[0, 0])
        d_ref[...] = d_fin
        pooled_ref[...] = jnp.where(d_fin > 0.0, acc_s[...] / d_fin, 0.0)


def _make_sc_attn(n_pad, nw, rows):
    nv = rows // 16
    mesh = plsc.VectorSubcoreMesh(core_axis_name="c", subcore_axis_name="s")

    @functools.partial(
        pl.kernel,
        mesh=mesh,
        compiler_params=pltpu.CompilerParams(needs_layout_passes=False),
        out_type=jax.ShapeDtypeStruct((n_pad,), jnp.float32),
        scratch_types=[
            pltpu.VMEM((rows,), jnp.float32),
            pltpu.VMEM((rows,), jnp.int32),
            pltpu.VMEM((NSEG,), jnp.float32),
            pltpu.VMEM((NSEG,), jnp.float32),
            pltpu.VMEM((rows,), jnp.float32),
            pltpu.SemaphoreType.DMA,
        ],
    )
    def sc_attn(s_hbm, b_hbm, m_hbm, d_hbm, out_hbm, s_v, b_v, m_v, d_v, a_v,
                sem):
        wid = lax.axis_index("s") * 2 + lax.axis_index("c")
        base = wid * rows
        c1 = pltpu.async_copy(s_hbm.at[pl.ds(base, rows)], s_v, sem)
        c2 = pltpu.async_copy(b_hbm.at[pl.ds(base, rows)], b_v, sem)
        c3 = pltpu.async_copy(m_hbm, m_v, sem)
        c4 = pltpu.async_copy(d_hbm, d_v, sem)
        c1.wait()
        c2.wait()
        c3.wait()
        c4.wait()

        msplat = m_v[pl.ds(0, 16)]
        for k in range(NSEG // 16):
            d_v[pl.ds(k * 16, 16)] = 1.0 / d_v[pl.ds(k * 16, 16)]

        def body(j, carry):
            for k in range(4):
                off = j * 64 + k * 16
                idx = b_v[pl.ds(off, 16)]
                sv = s_v[pl.ds(off, 16)]
                iv = plsc.load_gather(d_v, [idx])
                a_v[pl.ds(off, 16)] = jnp.exp(sv - msplat) * iv
            return carry

        lax.fori_loop(0, nv // 4, body, 0)
        pltpu.sync_copy(a_v, out_hbm.at[pl.ds(base, rows)])

    return sc_attn


def kernel(x, u, W1, b1, W2, b2, batch):
    n, nf = x.shape
    hid = W1.shape[1]
    batch = batch.astype(jnp.int32)

    B = 20000
    H = B // 2
    nsteps = n // B
    assert nsteps * B == n

    batch3 = batch.reshape(nsteps, 2, H)
    bh = batch3[:, :, 0]
    bl = batch3[:, :, H - 1]
    bounds = jnp.stack([bh, bl], axis=2)
    s3, m, d, pooled = pl.pallas_call(
        _main_body,
        grid=(nsteps,),
        in_specs=[
            pl.BlockSpec((B, nf), lambda i: (i, 0)),
            pl.BlockSpec((NSEG, nf), lambda i: (0, 0)),
            pl.BlockSpec((nf + nf, hid), lambda i: (0, 0)),
            pl.BlockSpec((1, hid), lambda i: (0, 0)),
            pl.BlockSpec((hid, 1), lambda i: (0, 0)),
            pl.BlockSpec((1, 1), lambda i: (0, 0)),
            pl.BlockSpec((1, 2, H), lambda i: (i, 0, 0)),
            pl.BlockSpec((1, 2, 2), lambda i: (i, 0, 0),
                         memory_space=pltpu.SMEM),
        ],
        out_specs=[
            pl.BlockSpec((1, 2, H), lambda i: (i, 0, 0)),
            pl.BlockSpec((1, NSEG), lambda i: (0, 0)),
            pl.BlockSpec((NSEG, 1), lambda i: (0, 0)),
            pl.BlockSpec((NSEG, nf), lambda i: (0, 0)),
        ],
        out_shape=[
            jax.ShapeDtypeStruct((nsteps, 2, H), jnp.float32),
            jax.ShapeDtypeStruct((1, NSEG), jnp.float32),
            jax.ShapeDtypeStruct((NSEG, 1), jnp.float32),
            jax.ShapeDtypeStruct((NSEG, nf), jnp.float32),
        ],
        scratch_shapes=[
            pltpu.SMEM((1, 1), jnp.float32),
            pltpu.VMEM((NSEG, 1), jnp.float32),
            pltpu.VMEM((NSEG, nf), jnp.float32),
            pltpu.VMEM((NSEG, hid), jnp.float32),
        ],
        compiler_params=pltpu.CompilerParams(
            dimension_semantics=("arbitrary",)),
    )(x, u, W1, b1.reshape(1, hid), W2, b2.reshape(1, 1), batch3, bounds)

    nw = 32
    rows = -(-n // (nw * 64)) * 64
    n_pad = rows * nw
    s_flat = s3.reshape(n)
    s_pad = jnp.pad(s_flat, (0, n_pad - n))
    b_pad = jnp.pad(batch, (0, n_pad - n))
    sc_attn = _make_sc_attn(n_pad, nw, rows)
    attn_pad = sc_attn(s_pad, b_pad, m.reshape(NSEG), d.reshape(NSEG))
    attn = attn_pad[:n]
    return pooled, attn
